# baseline (speedup 1.0000x reference)
import jax
import jax.numpy as jnp
from jax import lax
from jax.experimental import pallas as pl
from jax.experimental.pallas import tpu as pltpu
from jax.experimental.pallas import tpu_sc as plsc

N = 10000
E = 320000
D = 128
H = 128
C = 64

NC, NS = 2, 16
NW = NC * NS
K2 = 50
CPT2 = E // (NW * K2)
NP = 10112
RPT = NP // NS
KS = 20
CPTS = E // (NW * KS)
GS = 10
SSUP = CPTS // GS
GE = 5
ESUP = CPT2 // (2 * GE)
GD = 25
DSUP = CPT2 // GD


def _mesh():
    return plsc.VectorSubcoreMesh(core_axis_name="c", subcore_axis_name="s")


_UNTILED = pltpu.CompilerParams(use_tc_tiling_on_sc=False)


def _deg_body(i01_hbm, cnt_hbm, acc, idx_v, ones_v, zrow_v, sem):
    c = lax.axis_index("c")
    s = lax.axis_index("s")
    wid = c * NS + s

    def fill(r, carry):
        ones_v[r, :] = jnp.full((16,), 1.0, jnp.float32)
        zrow_v[r, :] = jnp.zeros((16,), jnp.float32)
        return carry

    lax.fori_loop(0, K2, fill, 0)
    for b in range(RPT // K2):
        pltpu.sync_copy(zrow_v, acc.at[pl.ds(s * RPT + b * K2, K2)])
    pltpu.sync_copy(zrow_v.at[pl.ds(0, RPT % K2)],
                    acc.at[pl.ds(s * RPT + (RPT // K2) * K2, RPT % K2)])
    pltpu.sync_copy(i01_hbm.at[0, wid], idx_v)
    plsc.subcore_barrier()

    def step(b, carry):
        descs = [
            pltpu.async_copy(ones_v, acc.at[idx_v.at[b * GD + j]], sem, add=True)
            for j in range(GD)
        ]
        for d in descs:
            d.wait()
        return carry

    lax.fori_loop(0, DSUP, step, 0)
    plsc.subcore_barrier()
    pltpu.sync_copy(acc.at[pl.ds(s * RPT, RPT)],
                    cnt_hbm.at[pl.ds(c * NP + s * RPT, RPT)])


def _deg_call(i01):
    fn = pl.kernel(
        _deg_body,
        out_type=jax.ShapeDtypeStruct((NC * NP, 16), jnp.float32),
        mesh=_mesh(),
        compiler_params=_UNTILED,
        scratch_types=[
            pltpu.VMEM_SHARED((NP, 16), jnp.float32),
            pltpu.VMEM((CPT2, K2), jnp.int32),
            pltpu.VMEM((K2, 16), jnp.float32),
            pltpu.VMEM((K2, 16), jnp.float32),
            pltpu.SemaphoreType.DMA,
        ],
    )
    return fn(i01)


def _spmm_body(lin_hbm, i01_hbm, out_hbm, acc, i0_v, i1_v, rows, gsem, ssem):
    c = lax.axis_index("c")
    s = lax.axis_index("s")
    wid = c * NS + s
    ZR = GS * KS

    def fill(r, carry):
        for q in range(H // 16):
            rows[r, pl.ds(q * 16, 16)] = jnp.zeros((16,), jnp.float32)
        return carry

    lax.fori_loop(0, ZR, fill, 0)
    for b in range(RPT // ZR):
        pltpu.sync_copy(rows, acc.at[pl.ds(s * RPT + b * ZR, ZR)])
    rem = RPT % ZR
    if rem:
        pltpu.sync_copy(rows.at[pl.ds(0, rem)],
                        acc.at[pl.ds(s * RPT + (RPT // ZR) * ZR, rem)])
    pltpu.sync_copy(i01_hbm.at[0, wid], i0_v)
    pltpu.sync_copy(i01_hbm.at[1, wid], i1_v)
    plsc.subcore_barrier()

    def step(ss, carry):
        gd = [
            pltpu.async_copy(lin_hbm.at[i0_v.at[ss * GS + g]],
                             rows.at[pl.ds(g * KS, KS)], gsem)
            for g in range(GS)
        ]
        sd = []
        for g in range(GS):
            gd[g].wait()
            sd.append(
                pltpu.async_copy(rows.at[pl.ds(g * KS, KS)],
                                 acc.at[i1_v.at[ss * GS + g]], ssem, add=True))
        for d in sd:
            d.wait()
        return carry

    lax.fori_loop(0, SSUP, step, 0)
    plsc.subcore_barrier()
    pltpu.sync_copy(acc.at[pl.ds(s * RPT, RPT)],
                    out_hbm.at[pl.ds(c * NP + s * RPT, RPT)])


def _spmm_call(lin, i01):
    fn = pl.kernel(
        _spmm_body,
        out_type=jax.ShapeDtypeStruct((NC * NP, H), jnp.float32),
        mesh=_mesh(),
        compiler_params=_UNTILED,
        scratch_types=[
            pltpu.VMEM_SHARED((NP, H), jnp.float32),
            pltpu.VMEM((CPTS, KS), jnp.int32),
            pltpu.VMEM((CPTS, KS), jnp.int32),
            pltpu.VMEM((GS * KS, H), jnp.float32),
            pltpu.SemaphoreType.DMA,
            pltpu.SemaphoreType.DMA,
        ],
    )
    return fn(lin, i01)


def _edge_body(p0_hbm, p1_hbm, i01_hbm, x_hbm,
               i0_v, i1_v, buf, gsem0, gsem1, wsem0, wsem1):
    c = lax.axis_index("c")
    s = lax.axis_index("s")
    wid = c * NS + s
    r0 = wid * CPT2
    gsem = (gsem0, gsem1)
    wsem = (wsem0, wsem1)
    pltpu.sync_copy(i01_hbm.at[0, wid], i0_v)
    pltpu.sync_copy(i01_hbm.at[1, wid], i1_v)

    def fire_p0(ss, a):
        for g in range(GE):
            pltpu.async_copy(p0_hbm.at[i0_v.at[ss * GE + g]], buf.at[a, g],
                             gsem[a])

    def fire_p1(ss, a):
        for g in range(GE):
            pltpu.async_copy(p1_hbm.at[i1_v.at[ss * GE + g]], buf.at[a, g],
                             gsem[a], add=True)

    def wait_g(a):
        for g in range(GE):
            pltpu.make_async_copy(p0_hbm.at[pl.ds(0, K2)], buf.at[a, g],
                                  gsem[a]).wait()

    def fire_wb(ss, a):
        for g in range(GE):
            pltpu.async_copy(buf.at[a, g],
                             x_hbm.at[pl.ds((r0 + ss * GE + g) * K2, K2)],
                             wsem[a])

    def drain_wb(a):
        for g in range(GE):
            pltpu.make_async_copy(p0_hbm.at[pl.ds(0, K2)], buf.at[a, g],
                                  wsem[a]).wait()

    fire_p0(0, 0)

    def step(i, carry):
        ss0 = 2 * i
        wait_g(0)
        fire_p1(ss0, 0)

        @pl.when(i > 0)
        def _():
            drain_wb(1)

        fire_p0(ss0 + 1, 1)
        wait_g(0)
        fire_wb(ss0, 0)
        wait_g(1)
        fire_p1(ss0 + 1, 1)
        drain_wb(0)
        wait_g(1)
        fire_wb(ss0 + 1, 1)

        @pl.when(i < ESUP - 1)
        def _():
            fire_p0(ss0 + 2, 0)

        return carry

    lax.fori_loop(0, ESUP, step, 0)
    drain_wb(1)


def _edge_call(p0, p1, i01):
    fn = pl.kernel(
        _edge_body,
        out_type=jax.ShapeDtypeStruct((E, C), jnp.float32),
        mesh=_mesh(),
        compiler_params=_UNTILED,
        scratch_types=[
            pltpu.VMEM((CPT2, K2), jnp.int32),
            pltpu.VMEM((CPT2, K2), jnp.int32),
            pltpu.VMEM((2, GE, K2, C), jnp.float32),
            pltpu.SemaphoreType.DMA,
            pltpu.SemaphoreType.DMA,
            pltpu.SemaphoreType.DMA,
            pltpu.SemaphoreType.DMA,
        ],
    )
    return fn(p0, p1, i01)


def _matT(x, w):
    return lax.dot_general(x, w, (((1,), (1,)), ((), ())),
                           preferred_element_type=jnp.float32)


def _pre_body(cnt_ref, feat_ref, w1_ref, b1_ref, dis_ref, lin_ref):
    deg = cnt_ref[:N, 0:1] + cnt_ref[NP:NP + N, 0:1] + 1.0
    dis = lax.rsqrt(deg)
    dis_ref[...] = dis
    lin_ref[...] = dis * (_matT(feat_ref[...], w1_ref[...]) + b1_ref[...])


def _mid_body(acc_ref, lin1_ref, dis_ref, w2_ref, b2_ref, lin2_ref):
    dis = dis_ref[...]
    h1 = dis * (acc_ref[:N, :] + acc_ref[NP:NP + N, :] + lin1_ref[...])
    lin2_ref[...] = dis * (_matT(h1, w2_ref[...]) + b2_ref[...])


def _post_body(acc_ref, lin2_ref, dis_ref, wfc0_ref, wfc1_ref, bfc_ref,
               p0_ref, p1_ref):
    dis = dis_ref[...]
    h2 = dis * (acc_ref[:N, :] + acc_ref[NP:NP + N, :] + lin2_ref[...])
    p0_ref[...] = _matT(h2, wfc0_ref[...]) + bfc_ref[...]
    p1_ref[...] = _matT(h2, wfc1_ref[...])


_BE = 8000


def _nll_sum(x, lab):
    m = jnp.max(x, axis=1, keepdims=True)
    lse = jnp.log(jnp.sum(jnp.exp(x - m), axis=1, keepdims=True)) + m
    onehot = lax.broadcasted_iota(jnp.int32, x.shape, 1) == lab
    xl = jnp.sum(jnp.where(onehot, x, 0.0), axis=1, keepdims=True)
    return jnp.sum(lse - xl)


def _loss_body(x2_ref, lab2_ref, out_ref):
    i = pl.program_id(0)
    x2 = x2_ref[...]
    lab2 = lab2_ref[...]
    part = (_nll_sum(x2[:, :C], lab2[:, 0:1]) +
            _nll_sum(x2[:, C:], lab2[:, 1:2]))

    @pl.when(i == 0)
    def _():
        out_ref[...] = jnp.zeros_like(out_ref)

    out_ref[...] += jnp.reshape(part, (1, 1))


def kernel(index, label, sentence_mask, features, edges, W1, b1, W2, b2, Wfc, bfc):
    i01 = index.astype(jnp.int32).reshape(2, NW, CPT2, K2)
    i01s = index.astype(jnp.int32).reshape(2, NW, CPTS, KS)

    cnt = _deg_call(i01)

    dis, lin1 = pl.pallas_call(
        _pre_body,
        out_shape=[jax.ShapeDtypeStruct((N, 1), jnp.float32),
                   jax.ShapeDtypeStruct((N, H), jnp.float32)],
    )(cnt, features, W1, b1.reshape(1, H))

    acc1 = _spmm_call(lin1, i01s)

    lin2 = pl.pallas_call(
        _mid_body,
        out_shape=jax.ShapeDtypeStruct((N, H), jnp.float32),
    )(acc1, lin1, dis, W2, b2.reshape(1, H))

    acc2 = _spmm_call(lin2, i01s)

    p0, p1 = pl.pallas_call(
        _post_body,
        out_shape=[jax.ShapeDtypeStruct((N, C), jnp.float32),
                   jax.ShapeDtypeStruct((N, C), jnp.float32)],
    )(acc2, lin2, dis, Wfc[:, :H], Wfc[:, H:], bfc.reshape(1, C))

    x = _edge_call(p0, p1, i01)

    x2 = x.reshape(E // 2, 2 * C)
    tot = pl.pallas_call(
        _loss_body,
        grid=(E // _BE,),
        in_specs=[pl.BlockSpec((_BE // 2, 2 * C), lambda i: (i, 0)),
                  pl.BlockSpec((_BE // 2, 2), lambda i: (i, 0))],
        out_specs=pl.BlockSpec((1, 1), lambda i: (0, 0)),
        out_shape=jax.ShapeDtypeStruct((1, 1), jnp.float32),
    )(x2, label.astype(jnp.int32).reshape(E // 2, 2))

    loss = tot[0, 0] / jnp.float32(E)
    return (loss, x)


# speedup vs baseline: 1.3051x; 1.3051x over previous
import jax
import jax.numpy as jnp
from jax import lax
from jax.experimental import pallas as pl
from jax.experimental.pallas import tpu as pltpu
from jax.experimental.pallas import tpu_sc as plsc

N = 10000
E = 320000
D = 128
H = 128
C = 64

NC, NS = 2, 16
NW = NC * NS
K2 = 50
CPT2 = E // (NW * K2)
NP = 10112
RPT = NP // NS
KS = 40
CPTS = E // (NW * KS)
GS = 5
SSUP = CPTS // GS
GE = 5
ESUP = CPT2 // (2 * GE)
GD = 25
DSUP = CPT2 // GD


def _mesh():
    return plsc.VectorSubcoreMesh(core_axis_name="c", subcore_axis_name="s")


_UNTILED = pltpu.CompilerParams(use_tc_tiling_on_sc=False)


def _deg_body(i01_hbm, cnt_hbm, acc, idx_v, ones_v, zrow_v, sem):
    c = lax.axis_index("c")
    s = lax.axis_index("s")
    wid = c * NS + s

    def fill(r, carry):
        ones_v[r, :] = jnp.full((16,), 1.0, jnp.float32)
        zrow_v[r, :] = jnp.zeros((16,), jnp.float32)
        return carry

    lax.fori_loop(0, K2, fill, 0)
    for b in range(RPT // K2):
        pltpu.sync_copy(zrow_v, acc.at[pl.ds(s * RPT + b * K2, K2)])
    pltpu.sync_copy(zrow_v.at[pl.ds(0, RPT % K2)],
                    acc.at[pl.ds(s * RPT + (RPT // K2) * K2, RPT % K2)])
    pltpu.sync_copy(i01_hbm.at[0, wid], idx_v)
    plsc.subcore_barrier()

    def step(b, carry):
        descs = [
            pltpu.async_copy(ones_v, acc.at[idx_v.at[b * GD + j]], sem, add=True)
            for j in range(GD)
        ]
        for d in descs:
            d.wait()
        return carry

    lax.fori_loop(0, DSUP, step, 0)
    plsc.subcore_barrier()
    pltpu.sync_copy(acc.at[pl.ds(s * RPT, RPT)],
                    cnt_hbm.at[pl.ds(c * NP + s * RPT, RPT)])


def _deg_call(i01):
    fn = pl.kernel(
        _deg_body,
        out_type=jax.ShapeDtypeStruct((NC * NP, 16), jnp.float32),
        mesh=_mesh(),
        compiler_params=_UNTILED,
        scratch_types=[
            pltpu.VMEM_SHARED((NP, 16), jnp.float32),
            pltpu.VMEM((CPT2, K2), jnp.int32),
            pltpu.VMEM((K2, 16), jnp.float32),
            pltpu.VMEM((K2, 16), jnp.float32),
            pltpu.SemaphoreType.DMA,
        ],
    )
    return fn(i01)


def _spmm_body(lin_hbm, i01_hbm, out_hbm, acc, i0_v, i1_v, rows, gsem, ssem):
    c = lax.axis_index("c")
    s = lax.axis_index("s")
    wid = c * NS + s
    ZR = GS * KS

    def fill(r, carry):
        for q in range(H // 16):
            rows[r, pl.ds(q * 16, 16)] = jnp.zeros((16,), jnp.float32)
        return carry

    lax.fori_loop(0, ZR, fill, 0)
    for b in range(RPT // ZR):
        pltpu.sync_copy(rows, acc.at[pl.ds(s * RPT + b * ZR, ZR)])
    rem = RPT % ZR
    if rem:
        pltpu.sync_copy(rows.at[pl.ds(0, rem)],
                        acc.at[pl.ds(s * RPT + (RPT // ZR) * ZR, rem)])
    pltpu.sync_copy(i01_hbm.at[0, wid], i0_v)
    pltpu.sync_copy(i01_hbm.at[1, wid], i1_v)
    plsc.subcore_barrier()

    def step(ss, carry):
        gd = [
            pltpu.async_copy(lin_hbm.at[i0_v.at[ss * GS + g]],
                             rows.at[pl.ds(g * KS, KS)], gsem)
            for g in range(GS)
        ]
        sd = []
        for g in range(GS):
            gd[g].wait()
            sd.append(
                pltpu.async_copy(rows.at[pl.ds(g * KS, KS)],
                                 acc.at[i1_v.at[ss * GS + g]], ssem, add=True))
        for d in sd:
            d.wait()
        return carry

    lax.fori_loop(0, SSUP, step, 0)
    plsc.subcore_barrier()
    pltpu.sync_copy(acc.at[pl.ds(s * RPT, RPT)],
                    out_hbm.at[pl.ds(c * NP + s * RPT, RPT)])


def _spmm_call(lin, i01):
    fn = pl.kernel(
        _spmm_body,
        out_type=jax.ShapeDtypeStruct((NC * NP, H), jnp.float32),
        mesh=_mesh(),
        compiler_params=_UNTILED,
        scratch_types=[
            pltpu.VMEM_SHARED((NP, H), jnp.float32),
            pltpu.VMEM((CPTS, KS), jnp.int32),
            pltpu.VMEM((CPTS, KS), jnp.int32),
            pltpu.VMEM((GS * KS, H), jnp.float32),
            pltpu.SemaphoreType.DMA,
            pltpu.SemaphoreType.DMA,
        ],
    )
    return fn(lin, i01)


def _edge_body(p0_hbm, p1_hbm, i01_hbm, x_hbm,
               i0_v, i1_v, buf, gsem0, gsem1, wsem0, wsem1):
    c = lax.axis_index("c")
    s = lax.axis_index("s")
    wid = c * NS + s
    r0 = s * CPT2
    gsem = (gsem0, gsem1)
    wsem = (wsem0, wsem1)
    pltpu.sync_copy(i01_hbm.at[0, wid], i0_v)
    pltpu.sync_copy(i01_hbm.at[1, wid], i1_v)

    def fire_p0(ss, a):
        for g in range(GE):
            pltpu.async_copy(p0_hbm.at[i0_v.at[ss * GE + g]], buf.at[a, g],
                             gsem[a])

    def fire_p1(ss, a):
        for g in range(GE):
            pltpu.async_copy(p1_hbm.at[i1_v.at[ss * GE + g]], buf.at[a, g],
                             gsem[a], add=True)

    def wait_g(a):
        for g in range(GE):
            pltpu.make_async_copy(p0_hbm.at[pl.ds(0, K2)], buf.at[a, g],
                                  gsem[a]).wait()

    def fire_wb(ss, a):
        for g in range(GE):
            pltpu.async_copy(buf.at[a, g],
                             x_hbm.at[pl.ds((r0 + ss * GE + g) * K2, K2),
                                      pl.ds(c * C, C)],
                             wsem[a])

    def drain_wb(a):
        for g in range(GE):
            pltpu.make_async_copy(p0_hbm.at[pl.ds(0, K2)], buf.at[a, g],
                                  wsem[a]).wait()

    fire_p0(0, 0)

    def step(i, carry):
        ss0 = 2 * i
        wait_g(0)
        fire_p1(ss0, 0)

        @pl.when(i > 0)
        def _():
            drain_wb(1)

        fire_p0(ss0 + 1, 1)
        wait_g(0)
        fire_wb(ss0, 0)
        wait_g(1)
        fire_p1(ss0 + 1, 1)
        drain_wb(0)
        wait_g(1)
        fire_wb(ss0 + 1, 1)

        @pl.when(i < ESUP - 1)
        def _():
            fire_p0(ss0 + 2, 0)

        return carry

    lax.fori_loop(0, ESUP, step, 0)
    drain_wb(1)


def _edge_call(p0, p1, i01):
    fn = pl.kernel(
        _edge_body,
        out_type=jax.ShapeDtypeStruct((E // 2, 2 * C), jnp.float32),
        mesh=_mesh(),
        compiler_params=_UNTILED,
        scratch_types=[
            pltpu.VMEM((CPT2, K2), jnp.int32),
            pltpu.VMEM((CPT2, K2), jnp.int32),
            pltpu.VMEM((2, GE, K2, C), jnp.float32),
            pltpu.SemaphoreType.DMA,
            pltpu.SemaphoreType.DMA,
            pltpu.SemaphoreType.DMA,
            pltpu.SemaphoreType.DMA,
        ],
    )
    return fn(p0, p1, i01)


def _matT(x, w):
    return lax.dot_general(x, w, (((1,), (1,)), ((), ())),
                           preferred_element_type=jnp.float32)


def _pre_body(cnt_ref, feat_ref, w1_ref, b1_ref, dis_ref, lin_ref):
    deg = cnt_ref[:N, 0:1] + cnt_ref[NP:NP + N, 0:1] + 1.0
    dis = lax.rsqrt(deg)
    dis_ref[...] = dis
    lin_ref[...] = dis * (_matT(feat_ref[...], w1_ref[...]) + b1_ref[...])


def _mid_body(acc_ref, lin1_ref, dis_ref, w2_ref, b2_ref, lin2_ref):
    dis = dis_ref[...]
    h1 = dis * (acc_ref[:N, :] + acc_ref[NP:NP + N, :] + lin1_ref[...])
    lin2_ref[...] = dis * (_matT(h1, w2_ref[...]) + b2_ref[...])


def _post_body(acc_ref, lin2_ref, dis_ref, wfc0_ref, wfc1_ref, bfc_ref,
               p0_ref, p1_ref):
    dis = dis_ref[...]
    h2 = dis * (acc_ref[:N, :] + acc_ref[NP:NP + N, :] + lin2_ref[...])
    p0_ref[...] = _matT(h2, wfc0_ref[...]) + bfc_ref[...]
    p1_ref[...] = _matT(h2, wfc1_ref[...])


_BE = 8000
_NBL = (E // 2) // _BE


def _ce_half(x, lab, xout_ref, loss_ref):
    xout_ref[...] = x
    m = jnp.max(x, axis=1, keepdims=True)
    ex = jnp.exp(x - m)
    ones = jnp.ones((C, 1), jnp.float32)
    s = lax.dot_general(ex, ones, (((1,), (0,)), ((), ())),
                        preferred_element_type=jnp.float32)
    onehot = lax.broadcasted_iota(jnp.int32, x.shape, 1) == lab
    xsel = jnp.where(onehot, x, 0.0)
    xl = lax.dot_general(xsel, ones, (((1,), (0,)), ((), ())),
                         preferred_element_type=jnp.float32)
    part = jnp.sum(jnp.log(s) + m - xl)
    loss_ref[...] += jnp.reshape(part, (1, 1))


def _loss_body(xc_ref, lab_ref, xout_ref, loss_ref):
    i = pl.program_id(0)
    j = pl.program_id(1)
    xf = xc_ref[...]
    lab = lab_ref[...]

    @pl.when((j == 0) & (i == 0))
    def _():
        loss_ref[...] = jnp.zeros_like(loss_ref)

    @pl.when(j == 0)
    def _():
        _ce_half(xf[:, :C], lab, xout_ref, loss_ref)

    @pl.when(j == 1)
    def _():
        _ce_half(xf[:, C:], lab, xout_ref, loss_ref)


def kernel(index, label, sentence_mask, features, edges, W1, b1, W2, b2, Wfc, bfc):
    i01 = index.astype(jnp.int32).reshape(2, NW, CPT2, K2)
    i01s = index.astype(jnp.int32).reshape(2, NW, CPTS, KS)

    cnt = _deg_call(i01)

    dis, lin1 = pl.pallas_call(
        _pre_body,
        out_shape=[jax.ShapeDtypeStruct((N, 1), jnp.float32),
                   jax.ShapeDtypeStruct((N, H), jnp.float32)],
    )(cnt, features, W1, b1.reshape(1, H))

    acc1 = _spmm_call(lin1, i01s)

    lin2 = pl.pallas_call(
        _mid_body,
        out_shape=jax.ShapeDtypeStruct((N, H), jnp.float32),
    )(acc1, lin1, dis, W2, b2.reshape(1, H))

    acc2 = _spmm_call(lin2, i01s)

    p0, p1 = pl.pallas_call(
        _post_body,
        out_shape=[jax.ShapeDtypeStruct((N, C), jnp.float32),
                   jax.ShapeDtypeStruct((N, C), jnp.float32)],
    )(acc2, lin2, dis, Wfc[:, :H], Wfc[:, H:], bfc.reshape(1, C))

    xc = _edge_call(p0, p1, i01)

    x, tot = pl.pallas_call(
        _loss_body,
        grid=(_NBL, 2),
        in_specs=[pl.BlockSpec((_BE, 2 * C), lambda i, j: (i, 0)),
                  pl.BlockSpec((_BE, 1), lambda i, j: (j * _NBL + i, 0))],
        out_specs=[pl.BlockSpec((_BE, C), lambda i, j: (j * _NBL + i, 0)),
                   pl.BlockSpec((1, 1), lambda i, j: (0, 0))],
        out_shape=[jax.ShapeDtypeStruct((E, C), jnp.float32),
                   jax.ShapeDtypeStruct((1, 1), jnp.float32)],
    )(xc, label.astype(jnp.int32).reshape(E, 1))

    loss = tot[0, 0] / jnp.float32(E)
    return (loss, x)
